# baseline (speedup 1.0000x reference)
import jax
import jax.numpy as jnp
import numpy as np
from jax.experimental import pallas as pl
from jax.experimental.pallas import tpu as pltpu
from jax.sharding import PartitionSpec as P

B, D, C, P_ = 2048, 128, 200, 200
N = B + P_
TILE = 768
NP_ = 2304
NT = NP_ // TILE
NPAD = NP_ - N
MARGIN = 1.0
N_PAIRS = float(N * (N - 1) // 2)

_CORE_TILES = np.array(
    [[[0, 0], [0, 1], [0, 2]],
     [[1, 1], [1, 2], [2, 2]]], dtype=np.int32)
_CORE_TILES = np.transpose(_CORE_TILES, (0, 2, 1)).copy()
TILES_PER_CORE = _CORE_TILES.shape[2]


def _body(tiles_ref, emb_ref, lab_ref, proto_ref, pk_ref,
          out_ref, xall, x2all, sq_c, sq_r, lab_c, lab_r):
    t = pl.program_id(0)

    @pl.when(t == 0)
    def _prep():
        out_ref[0, 0] = 0.0
        xall[0:B, :] = emb_ref[...]
        xall[B:N, :] = proto_ref[...]
        padv = 2.0 * (jax.lax.broadcasted_iota(jnp.int32, (NPAD, D), 0)
                      .astype(jnp.float32) + 1.0)
        xall[N:NP_, :] = padv
        x = xall[...]
        x2all[...] = x + x
        sq_c[...] = jnp.sum(x * x, axis=1, keepdims=True)
        v = lab_ref[...]
        m = jnp.max(v, axis=1, keepdims=True)
        iota = jax.lax.broadcasted_iota(jnp.int32, v.shape, 1)
        lab_c[0:B, :] = jnp.min(jnp.where(v == m, iota, C), axis=1,
                                keepdims=True)
        lab_c[B:N, :] = pk_ref[...]
        lab_c[N:NP_, :] = -(jax.lax.broadcasted_iota(jnp.int32, (NPAD, 1), 0)
                            + 1)
        for k in range(NT):
            sq_r[k:k + 1, :] = jnp.transpose(
                sq_c[k * TILE:(k + 1) * TILE, :])
            lab_r[k:k + 1, :] = jnp.transpose(
                lab_c[k * TILE:(k + 1) * TILE, :])

    bi = tiles_ref[0, t]
    bj = tiles_ref[1, t]
    ri = pl.ds(bi * TILE, TILE)
    rj = pl.ds(bj * TILE, TILE)

    xi = xall[ri, :]
    xj2 = x2all[rj, :]
    dot2 = jax.lax.dot_general(xi, xj2, (((1,), (1,)), ((), ())),
                               preferred_element_type=jnp.float32)
    sqi = sq_c[ri, :]
    sqj = sq_r[pl.ds(bj, 1), :]
    li = lab_c[ri, :]
    lj = lab_r[pl.ds(bj, 1), :]

    CH = 16
    eps = jnp.float32(1e-12)
    one = jnp.float32(MARGIN * MARGIN)
    acc = jnp.zeros((CH, TILE), jnp.float32)
    for k in range(TILE // CH):
        sl = slice(k * CH, (k + 1) * CH)
        raw = (sqi[sl, :] + sqj) - dot2[sl, :]
        q = jax.lax.clamp(eps, raw, one)
        r = MARGIN - q * jax.lax.rsqrt(q)
        same = li[sl, :] == lj
        acc = acc + jnp.where(same, raw, r * r)
    s = jnp.sum(acc)

    scale = jnp.where(bi == bj, 0.5, 1.0)
    out_ref[0, 0] += s * scale


def _core_fn(tl, embeddings, labels, prototypes, pk2d):
    out = pl.pallas_call(
        _body,
        grid_spec=pltpu.PrefetchScalarGridSpec(
            num_scalar_prefetch=1,
            grid=(TILES_PER_CORE,),
            in_specs=[
                pl.BlockSpec((B, D), lambda t, tiles: (0, 0)),
                pl.BlockSpec((B, C), lambda t, tiles: (0, 0)),
                pl.BlockSpec((P_, D), lambda t, tiles: (0, 0)),
                pl.BlockSpec((P_, 1), lambda t, tiles: (0, 0)),
            ],
            out_specs=pl.BlockSpec(memory_space=pltpu.SMEM),
            scratch_shapes=[
                pltpu.VMEM((NP_, D), jnp.float32),
                pltpu.VMEM((NP_, D), jnp.float32),
                pltpu.VMEM((NP_, 1), jnp.float32),
                pltpu.VMEM((NT, TILE), jnp.float32),
                pltpu.VMEM((NP_, 1), jnp.int32),
                pltpu.VMEM((NT, TILE), jnp.int32),
            ],
        ),
        out_shape=jax.ShapeDtypeStruct((1, 1), jnp.float32),
    )(tl, embeddings, labels, prototypes, pk2d)
    return out[0, 0]


def kernel(embeddings, labels, prototypes, proto_keys):
    pk2d = proto_keys.astype(jnp.int32)[:, None]
    core_tiles = jnp.asarray(_CORE_TILES)
    mesh = jax.make_mesh((2,), ("c",))

    def shard_fn(emb, lab, proto, pk):
        cid = jax.lax.axis_index("c")
        tl = jax.lax.dynamic_index_in_dim(core_tiles, cid, axis=0,
                                          keepdims=False)
        partial = _core_fn(tl, emb, lab, proto, pk)
        return jax.lax.psum(partial, "c")

    total = jax.shard_map(shard_fn, mesh=mesh,
                          in_specs=(P(), P(), P(), P()),
                          out_specs=P(), check_vma=False)(
        embeddings, labels, prototypes, pk2d)
    return total * (1.0 / N_PAIRS)


# speedup vs baseline: 10.0183x; 10.0183x over previous
import jax
import jax.numpy as jnp
import numpy as np
from jax.experimental import pallas as pl
from jax.experimental.pallas import tpu as pltpu

B, D, C, P = 2048, 128, 200, 200
N = B + P
TILE = 768
NP_ = 2304
NT = NP_ // TILE
NPAD = NP_ - N
MARGIN = 1.0
N_PAIRS = float(N * (N - 1) // 2)

_PAIRS = np.array([(i, j) for i in range(NT) for j in range(i, NT)],
                  dtype=np.int32).T
NUM_TILES = _PAIRS.shape[1]
SUB = 256


def _masked_sum(dot2, sqi, sqj, li, lj):
    rows, cols = dot2.shape
    ch = 16
    eps = jnp.float32(1e-12)
    m2 = jnp.float32(MARGIN * MARGIN)
    acc0 = jnp.zeros((ch, cols), jnp.float32)
    acc1 = jnp.zeros((ch, cols), jnp.float32)
    for k in range(rows // ch):
        sl = slice(k * ch, (k + 1) * ch)
        raw = (sqi[sl, :] + sqj) - dot2[sl, :]
        q = jax.lax.clamp(eps, raw, m2)
        r = MARGIN - q * jax.lax.rsqrt(q)
        same = li[sl, :] == lj
        v = jnp.where(same, raw, r * r)
        if k % 2 == 0:
            acc0 = acc0 + v
        else:
            acc1 = acc1 + v
    return jnp.sum(acc0 + acc1)


def _body(tiles_ref, emb_ref, lab_ref, proto_ref, pk_ref,
          out_ref, xall, x2all, sq_c, sq_r, lab_c, lab_r):
    t = pl.program_id(0)

    @pl.when(t == 0)
    def _prep():
        out_ref[0, 0] = 0.0
        xall[0:B, :] = emb_ref[...]
        xall[B:N, :] = proto_ref[...]
        padv = 2.0 * (jax.lax.broadcasted_iota(jnp.int32, (NPAD, D), 0)
                      .astype(jnp.float32) + 1.0)
        xall[N:NP_, :] = padv
        x = xall[...]
        x2all[...] = x + x
        sq_c[...] = jnp.sum(x * x, axis=1, keepdims=True)
        iota = jax.lax.broadcasted_iota(jnp.int32, (128, C), 1)
        for k in range(B // 128):
            vk = lab_ref[k * 128:(k + 1) * 128, :]
            m = jnp.max(vk, axis=1, keepdims=True)
            lab_c[k * 128:(k + 1) * 128, :] = jnp.min(
                jnp.where(vk == m, iota, C), axis=1, keepdims=True)
        lab_c[B:N, :] = pk_ref[...]
        lab_c[N:NP_, :] = -(jax.lax.broadcasted_iota(jnp.int32, (NPAD, 1), 0)
                            + 1)
        for k in range(NT):
            sq_r[k:k + 1, :] = jnp.transpose(
                sq_c[k * TILE:(k + 1) * TILE, :])
            lab_r[k:k + 1, :] = jnp.transpose(
                lab_c[k * TILE:(k + 1) * TILE, :])

    bi = tiles_ref[0, t]
    bj = tiles_ref[1, t]
    ri = pl.ds(bi * TILE, TILE)
    rj = pl.ds(bj * TILE, TILE)

    xi = xall[ri, :]
    sqi = sq_c[ri, :]
    sqj = sq_r[pl.ds(bj, 1), :]
    li = lab_c[ri, :]
    lj = lab_r[pl.ds(bj, 1), :]

    @pl.when(bi != bj)
    def _offdiag():
        xj2 = x2all[rj, :]
        dot2 = jax.lax.dot_general(xi, xj2, (((1,), (1,)), ((), ())),
                                   preferred_element_type=jnp.float32)
        out_ref[0, 0] += _masked_sum(dot2, sqi, sqj, li, lj)

    @pl.when(bi == bj)
    def _diag():
        s = 0.0
        for a in range(TILE // SUB):
            sla = slice(a * SUB, (a + 1) * SUB)
            xia = xi[sla, :]
            for b in range(a, TILE // SUB):
                xj2b = x2all[pl.ds(bj * TILE + b * SUB, SUB), :]
                dot2 = jax.lax.dot_general(
                    xia, xj2b, (((1,), (1,)), ((), ())),
                    preferred_element_type=jnp.float32)
                sub = _masked_sum(
                    dot2, sqi[sla, :], sqj[:, b * SUB:(b + 1) * SUB],
                    li[sla, :], lj[:, b * SUB:(b + 1) * SUB])
                s = s + (0.5 * sub if a == b else sub)
        out_ref[0, 0] += s

    @pl.when(t == NUM_TILES - 1)
    def _finish():
        out_ref[0, 0] = out_ref[0, 0] * (1.0 / N_PAIRS)


def kernel(embeddings, labels, prototypes, proto_keys):
    tiles = jnp.asarray(_PAIRS)
    pk2d = proto_keys.astype(jnp.int32)[:, None]

    out = pl.pallas_call(
        _body,
        grid_spec=pltpu.PrefetchScalarGridSpec(
            num_scalar_prefetch=1,
            grid=(NUM_TILES,),
            in_specs=[
                pl.BlockSpec((B, D), lambda t, tiles: (0, 0)),
                pl.BlockSpec((B, C), lambda t, tiles: (0, 0)),
                pl.BlockSpec((P, D), lambda t, tiles: (0, 0)),
                pl.BlockSpec((P, 1), lambda t, tiles: (0, 0)),
            ],
            out_specs=pl.BlockSpec(memory_space=pltpu.SMEM),
            scratch_shapes=[
                pltpu.VMEM((NP_, D), jnp.float32),
                pltpu.VMEM((NP_, D), jnp.float32),
                pltpu.VMEM((NP_, 1), jnp.float32),
                pltpu.VMEM((NT, TILE), jnp.float32),
                pltpu.VMEM((NP_, 1), jnp.int32),
                pltpu.VMEM((NT, TILE), jnp.int32),
            ],
        ),
        out_shape=jax.ShapeDtypeStruct((1, 1), jnp.float32),
    )(tiles, embeddings, labels, prototypes, pk2d)
    return out[0, 0]


# speedup vs baseline: 11.0380x; 1.1018x over previous
import jax
import jax.numpy as jnp
import numpy as np
from jax.experimental import pallas as pl
from jax.experimental.pallas import tpu as pltpu

B, D, C, P = 2048, 128, 200, 200
N = B + P
TILE = 768
NP_ = 2304
NT = NP_ // TILE
NPAD = NP_ - N
MARGIN = 1.0
N_PAIRS = float(N * (N - 1) // 2)

_PAIRS = np.array([(i, j) for i in range(NT) for j in range(i, NT)],
                  dtype=np.int32).T
NUM_TILES = _PAIRS.shape[1]


_SQRT2 = np.float32(np.sqrt(2.0))


def _body(tiles_ref, emb_ref, lab_ref, proto_ref, pk_ref,
          out_ref, yall, sq_c, sq_r, lab_c, lab_r):
    t = pl.program_id(0)

    @pl.when(t == 0)
    def _prep():
        out_ref[0, 0] = 0.0
        yall[0:B, :] = emb_ref[...] * _SQRT2
        yall[B:N, :] = proto_ref[...] * _SQRT2
        padv = (2.0 * _SQRT2) * (
            jax.lax.broadcasted_iota(jnp.int32, (NPAD, D), 0)
            .astype(jnp.float32) + 1.0)
        yall[N:NP_, :] = padv
        y = yall[...]
        sq_c[...] = 0.5 * jnp.sum(y * y, axis=1, keepdims=True)
        v = lab_ref[...]
        m = jnp.max(v, axis=1, keepdims=True)
        iota = jax.lax.broadcasted_iota(jnp.int32, v.shape, 1)
        lab_c[0:B, :] = jnp.min(jnp.where(v == m, iota, C), axis=1,
                                keepdims=True)
        lab_c[B:N, :] = pk_ref[...]
        lab_c[N:NP_, :] = -(jax.lax.broadcasted_iota(jnp.int32, (NPAD, 1), 0)
                            + 1)
        for k in range(NT):
            sq_r[k:k + 1, :] = jnp.transpose(
                sq_c[k * TILE:(k + 1) * TILE, :])
            lab_r[k:k + 1, :] = jnp.transpose(
                lab_c[k * TILE:(k + 1) * TILE, :])

    bi = tiles_ref[0, t]
    bj = tiles_ref[1, t]
    ri = pl.ds(bi * TILE, TILE)
    rj = pl.ds(bj * TILE, TILE)

    yi = yall[ri, :]
    yj = yall[rj, :]
    dot2 = jax.lax.dot_general(yi, yj, (((1,), (1,)), ((), ())),
                               preferred_element_type=jnp.float32)
    sqi = sq_c[ri, :]
    sqj = sq_r[pl.ds(bj, 1), :]
    li = lab_c[ri, :]
    lj = lab_r[pl.ds(bj, 1), :]

    CH = 16
    eps = jnp.float32(1e-12)
    one = jnp.float32(MARGIN * MARGIN)
    acc = jnp.zeros((CH, TILE), jnp.float32)
    for k in range(TILE // CH):
        sl = slice(k * CH, (k + 1) * CH)
        raw = (sqi[sl, :] + sqj) - dot2[sl, :]
        q = jax.lax.clamp(eps, raw, one)
        r = MARGIN - q * jax.lax.rsqrt(q)
        same = li[sl, :] == lj
        acc = acc + jnp.where(same, raw, r * r)
    s = jnp.sum(acc)

    scale = jnp.where(bi == bj, 0.5, 1.0)
    out_ref[0, 0] += s * scale

    @pl.when(t == NUM_TILES - 1)
    def _finish():
        out_ref[0, 0] = out_ref[0, 0] * (1.0 / N_PAIRS)


def kernel(embeddings, labels, prototypes, proto_keys):
    tiles = jnp.asarray(_PAIRS)
    pk2d = proto_keys.astype(jnp.int32)[:, None]

    out = pl.pallas_call(
        _body,
        grid_spec=pltpu.PrefetchScalarGridSpec(
            num_scalar_prefetch=1,
            grid=(NUM_TILES,),
            in_specs=[
                pl.BlockSpec((B, D), lambda t, tiles: (0, 0)),
                pl.BlockSpec((B, C), lambda t, tiles: (0, 0)),
                pl.BlockSpec((P, D), lambda t, tiles: (0, 0)),
                pl.BlockSpec((P, 1), lambda t, tiles: (0, 0)),
            ],
            out_specs=pl.BlockSpec(memory_space=pltpu.SMEM),
            scratch_shapes=[
                pltpu.VMEM((NP_, D), jnp.float32),
                pltpu.VMEM((NP_, 1), jnp.float32),
                pltpu.VMEM((NT, TILE), jnp.float32),
                pltpu.VMEM((NP_, 1), jnp.int32),
                pltpu.VMEM((NT, TILE), jnp.int32),
            ],
        ),
        out_shape=jax.ShapeDtypeStruct((1, 1), jnp.float32),
    )(tiles, embeddings, labels, prototypes, pk2d)
    return out[0, 0]


# speedup vs baseline: 11.0964x; 1.0053x over previous
import jax
import jax.numpy as jnp
import numpy as np
from jax.experimental import pallas as pl
from jax.experimental.pallas import tpu as pltpu

B, D, C, P = 2048, 128, 200, 200
N = B + P
TILE = 768
NP_ = 2304
NT = NP_ // TILE
NPAD = NP_ - N
MARGIN = 1.0
N_PAIRS = float(N * (N - 1) // 2)

_PAIRS = np.array([(i, j) for i in range(NT) for j in range(i, NT)],
                  dtype=np.int32).T
NUM_TILES = _PAIRS.shape[1]


def _body(tiles_ref, emb_ref, lab_ref, proto_ref, pk_ref,
          out_ref, xall, x2all, sq_c, sq_r, lab_c, lab_r):
    t = pl.program_id(0)

    @pl.when(t == 0)
    def _prep():
        out_ref[0, 0] = 0.0
        xall[0:B, :] = emb_ref[...]
        xall[B:N, :] = proto_ref[...]
        padv = 2.0 * (jax.lax.broadcasted_iota(jnp.int32, (NPAD, D), 0)
                      .astype(jnp.float32) + 1.0)
        xall[N:NP_, :] = padv
        x = xall[...]
        x2all[...] = x + x
        sq_c[...] = jnp.sum(x * x, axis=1, keepdims=True)
        v = lab_ref[...]
        m = jnp.max(v, axis=1, keepdims=True)
        iota = jax.lax.broadcasted_iota(jnp.int32, v.shape, 1)
        lab_c[0:B, :] = jnp.min(jnp.where(v == m, iota, C), axis=1,
                                keepdims=True)
        lab_c[B:N, :] = pk_ref[...]
        lab_c[N:NP_, :] = -(jax.lax.broadcasted_iota(jnp.int32, (NPAD, 1), 0)
                            + 1)
        for k in range(NT):
            sq_r[k:k + 1, :] = jnp.transpose(
                sq_c[k * TILE:(k + 1) * TILE, :])
            lab_r[k:k + 1, :] = jnp.transpose(
                lab_c[k * TILE:(k + 1) * TILE, :])

    bi = tiles_ref[0, t]
    bj = tiles_ref[1, t]
    ri = pl.ds(bi * TILE, TILE)
    rj = pl.ds(bj * TILE, TILE)

    xi = xall[ri, :]
    xj2 = x2all[rj, :]
    dot2 = jax.lax.dot_general(xi, xj2, (((1,), (1,)), ((), ())),
                               preferred_element_type=jnp.float32)
    sqi = sq_c[ri, :]
    sqj = sq_r[pl.ds(bj, 1), :]
    li = lab_c[ri, :]
    lj = lab_r[pl.ds(bj, 1), :]

    CH = 16
    acc0 = jnp.zeros((CH, TILE), jnp.float32)
    acc1 = jnp.zeros((CH, TILE), jnp.float32)
    for k in range(TILE // CH):
        sl = slice(k * CH, (k + 1) * CH)
        raw = (sqi[sl, :] + sqj) - dot2[sl, :]
        q = jnp.clip(raw, 1e-12, MARGIN * MARGIN)
        r = MARGIN - q * jax.lax.rsqrt(q)
        same = li[sl, :] == lj
        v = jnp.where(same, raw, r * r)
        if k % 2 == 0:
            acc0 = acc0 + v
        else:
            acc1 = acc1 + v
    s = jnp.sum(acc0 + acc1)

    scale = jnp.where(bi == bj, 0.5, 1.0)
    out_ref[0, 0] += s * scale

    @pl.when(t == NUM_TILES - 1)
    def _finish():
        out_ref[0, 0] = out_ref[0, 0] * (1.0 / N_PAIRS)


def kernel(embeddings, labels, prototypes, proto_keys):
    tiles = jnp.asarray(_PAIRS)
    pk2d = proto_keys.astype(jnp.int32)[:, None]

    out = pl.pallas_call(
        _body,
        grid_spec=pltpu.PrefetchScalarGridSpec(
            num_scalar_prefetch=1,
            grid=(NUM_TILES,),
            in_specs=[
                pl.BlockSpec((B, D), lambda t, tiles: (0, 0)),
                pl.BlockSpec((B, C), lambda t, tiles: (0, 0)),
                pl.BlockSpec((P, D), lambda t, tiles: (0, 0)),
                pl.BlockSpec((P, 1), lambda t, tiles: (0, 0)),
            ],
            out_specs=pl.BlockSpec(memory_space=pltpu.SMEM),
            scratch_shapes=[
                pltpu.VMEM((NP_, D), jnp.float32),
                pltpu.VMEM((NP_, D), jnp.float32),
                pltpu.VMEM((NP_, 1), jnp.float32),
                pltpu.VMEM((NT, TILE), jnp.float32),
                pltpu.VMEM((NP_, 1), jnp.int32),
                pltpu.VMEM((NT, TILE), jnp.int32),
            ],
        ),
        out_shape=jax.ShapeDtypeStruct((1, 1), jnp.float32),
    )(tiles, embeddings, labels, prototypes, pk2d)
    return out[0, 0]
